# baseline (speedup 1.0000x reference)
import functools

import jax
import jax.numpy as jnp
from jax import lax
from jax.experimental import pallas as pl
from jax.experimental.pallas import tpu as pltpu
from jax.experimental.pallas import tpu_sc as plsc

_P_THRESHOLD = 0.1
_OVERLAP_THRESHOLD = 0.3
_N_MAX_OBJECTS = 20
_L = 16
_BIG = 2**30


def _sc_body(n_real, n_pad, n_batch,
             p_hbm, bx_hbm, by_hbm, bw_hbm, bh_hbm,
             op_hbm, ox_hbm, oy_hbm, ow_hbm, oh_hbm,
             s_v, e_v, bx_v, by_v, bw_v, bh_v,
             o_p, o_x, o_y, o_w, o_h):
    nchunks = n_pad // _L
    wid = lax.axis_index("s") * 2 + lax.axis_index("c")

    @pl.when(wid < n_batch)
    def _():
        b = wid
        pltpu.sync_copy(p_hbm.at[b], s_v)
        pltpu.sync_copy(bx_hbm.at[b], bx_v)
        pltpu.sync_copy(by_hbm.at[b], by_v)
        pltpu.sync_copy(bw_hbm.at[b], bw_v)
        pltpu.sync_copy(bh_hbm.at[b], bh_v)

        iota = lax.iota(jnp.int32, _L)

        def init(i, _):
            off = i * _L
            pc = s_v[pl.ds(off, _L)]
            s_v[pl.ds(off, _L)] = jnp.where(pc > _P_THRESHOLD, pc, 0.0)
            e_v[pl.ds(off, _L)] = jnp.where(iota + off >= n_real, 1.0, 0.0)
            return 0
        lax.fori_loop(0, nchunks, init, 0)

        def step(l, _):
            def scan_a(i, carry):
                vmax, varg, emin = carry
                off = i * _L
                sc = s_v[pl.ds(off, _L)]
                ec = e_v[pl.ds(off, _L)]
                idx = iota + off
                gt = sc > vmax
                vmax = jnp.where(gt, sc, vmax)
                varg = jnp.where(gt, idx, varg)
                emin = jnp.minimum(emin, jnp.where(ec == 0.0, idx, _BIG))
                return vmax, varg, emin
            vmax, varg, emin = lax.fori_loop(
                0, nchunks, scan_a,
                (jnp.zeros((_L,), jnp.float32),
                 jnp.full((_L,), _BIG, jnp.int32),
                 jnp.full((_L,), _BIG, jnp.int32)))

            dnums = lax.GatherDimensionNumbers(
                offset_dims=(), collapsed_slice_dims=(0,),
                start_index_map=(0,))

            def shuffle(v, sh):
                return lax.gather(
                    v, (iota ^ sh)[:, None], dnums, (1,),
                    mode=lax.GatherScatterMode.PROMISE_IN_BOUNDS)

            def xmax(v):
                for sh in (8, 4, 2, 1):
                    v = jnp.maximum(v, shuffle(v, sh))
                return v

            def xmin(v):
                for sh in (8, 4, 2, 1):
                    v = jnp.minimum(v, shuffle(v, sh))
                return v

            pmax = xmax(vmax)
            valid = pmax > 0.0
            m = xmin(jnp.where(vmax == pmax, varg, _BIG))
            m2 = xmin(emin)
            idxs = jnp.where(valid, m, m2)
            chosen = idxs[0]
            val = jnp.where(valid, pmax, 0.0)
            vf = jnp.where(valid, 1.0, 0.0)

            coff = (chosen // _L) * _L
            lanev = idxs - coff

            def bcast(ref):
                chunk = ref[pl.ds(coff, _L)]
                return lax.gather(
                    chunk, lanev[:, None], dnums, (1,),
                    mode=lax.GatherScatterMode.PROMISE_IN_BOUNDS)

            bxm = bcast(bx_v)
            bym = bcast(by_v)
            bwm = bcast(bw_v)
            bhm = bcast(bh_v)
            x1m = bxm - 0.5 * bwm
            x3m = bxm + 0.5 * bwm
            y1m = bym - 0.5 * bhm
            y3m = bym + 0.5 * bhm
            am = bwm * bhm

            def scan_b(i, _):
                off = i * _L
                bxc = bx_v[pl.ds(off, _L)]
                byc = by_v[pl.ds(off, _L)]
                bwc = bw_v[pl.ds(off, _L)]
                bhc = bh_v[pl.ds(off, _L)]
                sc = s_v[pl.ds(off, _L)]
                x1c = bxc - 0.5 * bwc
                x3c = bxc + 0.5 * bwc
                y1c = byc - 0.5 * bhc
                y3c = byc + 0.5 * bhc
                ac = bwc * bhc
                inter = (jnp.maximum(jnp.minimum(x3c, x3m) - jnp.maximum(x1c, x1m), 0.0)
                         * jnp.maximum(jnp.minimum(y3c, y3m) - jnp.maximum(y1c, y1m), 0.0))
                ov = jnp.where(inter / jnp.minimum(ac, am) > _OVERLAP_THRESHOLD,
                               1.0, 0.0)
                s_v[pl.ds(off, _L)] = sc * (1.0 - vf * ov)
                return 0
            lax.fori_loop(0, nchunks, scan_b, 0)

            eoff = (chosen // _L) * _L
            elane = chosen - eoff
            ec = e_v[pl.ds(eoff, _L)]
            e_v[pl.ds(eoff, _L)] = jnp.where(iota == elane, 1.0, ec)

            ooff = (l // _L) * _L
            olane = l - ooff
            at = iota == olane
            o_p[pl.ds(ooff, _L)] = jnp.where(at, val, o_p[pl.ds(ooff, _L)])
            o_x[pl.ds(ooff, _L)] = jnp.where(at, bxm, o_x[pl.ds(ooff, _L)])
            o_y[pl.ds(ooff, _L)] = jnp.where(at, bym, o_y[pl.ds(ooff, _L)])
            o_w[pl.ds(ooff, _L)] = jnp.where(at, bwm, o_w[pl.ds(ooff, _L)])
            o_h[pl.ds(ooff, _L)] = jnp.where(at, bhm, o_h[pl.ds(ooff, _L)])
            return 0

        lax.fori_loop(0, _N_MAX_OBJECTS, step, 0)

        pltpu.sync_copy(o_p, op_hbm.at[b])
        pltpu.sync_copy(o_x, ox_hbm.at[b])
        pltpu.sync_copy(o_y, oy_hbm.at[b])
        pltpu.sync_copy(o_w, ow_hbm.at[b])
        pltpu.sync_copy(o_h, oh_hbm.at[b])


@jax.jit
def kernel(prob, bx_dimfull, by_dimfull, bw_dimfull, bh_dimfull):
    b, n, _ = prob.shape
    n_pad = ((n + _L - 1) // _L) * _L
    n_out = 2 * _L

    def prep(v, fill):
        v = v[..., 0]
        return jnp.pad(v, ((0, 0), (0, n_pad - n)), constant_values=fill)

    p = prep(prob, 0.0)
    bx = prep(bx_dimfull, 0.0)
    by = prep(by_dimfull, 0.0)
    bw = prep(bw_dimfull, 1.0)
    bh = prep(bh_dimfull, 1.0)

    mesh = plsc.VectorSubcoreMesh(core_axis_name="c", subcore_axis_name="s")
    out = jax.ShapeDtypeStruct((b, n_out), jnp.float32)
    run = pl.kernel(
        functools.partial(_sc_body, n, n_pad, b),
        mesh=mesh,
        out_type=(out, out, out, out, out),
        scratch_types=[
            pltpu.VMEM((n_pad,), jnp.float32),
            pltpu.VMEM((n_pad,), jnp.float32),
            pltpu.VMEM((n_pad,), jnp.float32),
            pltpu.VMEM((n_pad,), jnp.float32),
            pltpu.VMEM((n_pad,), jnp.float32),
            pltpu.VMEM((n_pad,), jnp.float32),
            pltpu.VMEM((n_out,), jnp.float32),
            pltpu.VMEM((n_out,), jnp.float32),
            pltpu.VMEM((n_out,), jnp.float32),
            pltpu.VMEM((n_out,), jnp.float32),
            pltpu.VMEM((n_out,), jnp.float32),
        ],
    )
    ap, ax, ay, aw, ah = run(p, bx, by, bw, bh)

    k = min(_N_MAX_OBJECTS, n)
    return (ap[:, :k, None], ax[:, :k, None], ay[:, :k, None],
            aw[:, :k, None], ah[:, :k, None])


# speedup vs baseline: 4.8046x; 4.8046x over previous
import functools

import jax
import jax.numpy as jnp
from jax.experimental import pallas as pl
from jax.experimental.pallas import tpu as pltpu

_P_THRESHOLD = 0.1
_OVERLAP_THRESHOLD = 0.3
_N_MAX_OBJECTS = 20
_LANE = 128


def _nms_body(n_real, p_ref, bx_ref, by_ref, bw_ref, bh_ref,
              op_ref, ox_ref, oy_ref, ow_ref, oh_ref, s_ref, e_ref):
    p = p_ref[...]
    bx = bx_ref[...]
    by = by_ref[...]
    bw = bw_ref[...]
    bh = bh_ref[...]
    b, n = p.shape

    x1 = bx - 0.5 * bw
    x3 = bx + 0.5 * bw
    y1 = by - 0.5 * bh
    y3 = by + 0.5 * bh
    area = bw * bh

    col = jax.lax.broadcasted_iota(jnp.int32, (b, n), 1)
    slot = jax.lax.broadcasted_iota(jnp.int32, (b, _LANE), 1)

    s_ref[...] = p
    e_ref[...] = jnp.where(col >= n_real, 1.0, 0.0)
    zacc = jnp.zeros((b, _LANE), dtype=jnp.float32)
    op_ref[...] = zacc
    ox_ref[...] = zacc
    oy_ref[...] = zacc
    ow_ref[...] = zacc
    oh_ref[...] = zacc

    def body(l, _):
        s = s_ref[...]
        pmax = jnp.max(s, axis=1, keepdims=True)
        valid = pmax > 0.0
        vf = jnp.where(valid, 1.0, 0.0)
        m = jnp.min(jnp.where(s == pmax, col, n), axis=1, keepdims=True)
        sel = (col == m).astype(jnp.float32) * vf

        def pick(v):
            return jnp.sum(sel * v, axis=1, keepdims=True)

        bxm = pick(bx)
        bym = pick(by)
        bwm = pick(bw)
        bhm = pick(bh)

        x1m = bxm - 0.5 * bwm
        x3m = bxm + 0.5 * bwm
        y1m = bym - 0.5 * bhm
        y3m = bym + 0.5 * bhm
        aream = bwm * bhm
        inter = (jnp.maximum(jnp.minimum(x3, x3m) - jnp.maximum(x1, x1m), 0.0)
                 * jnp.maximum(jnp.minimum(y3, y3m) - jnp.maximum(y1, y1m), 0.0))
        ov = jnp.where(inter / jnp.minimum(area, aream) > _OVERLAP_THRESHOLD,
                       vf, 0.0)
        s_ref[...] = s * (1.0 - ov)
        e_ref[...] = jnp.maximum(e_ref[...], sel)

        at = slot == l
        op_ref[...] = jnp.where(at, jnp.where(valid, pmax, 0.0), op_ref[...])
        ox_ref[...] = jnp.where(at, bxm, ox_ref[...])
        oy_ref[...] = jnp.where(at, bym, oy_ref[...])
        ow_ref[...] = jnp.where(at, bwm, ow_ref[...])
        oh_ref[...] = jnp.where(at, bhm, oh_ref[...])
        return 0

    jax.lax.fori_loop(0, _N_MAX_OBJECTS, body, 0)

    used = jnp.where(slot < _N_MAX_OBJECTS, op_ref[...], 1.0)
    some_empty = jnp.min(used) == 0.0

    @pl.when(some_empty)
    def _fill():
        def fbody(l, _):
            at = slot == l
            cur = jnp.sum(jnp.where(at, op_ref[...], 0.0), axis=1,
                          keepdims=True)
            empty = (cur == 0.0).astype(jnp.float32)
            e = e_ref[...]
            m2 = jnp.min(jnp.where(e > 0.0, n, col), axis=1, keepdims=True)
            sel = (col == m2).astype(jnp.float32) * empty
            e_ref[...] = jnp.maximum(e, sel)

            def pick(v):
                return jnp.sum(sel * v, axis=1, keepdims=True)

            w = at & (empty > 0.0)
            ox_ref[...] = jnp.where(w, pick(bx), ox_ref[...])
            oy_ref[...] = jnp.where(w, pick(by), oy_ref[...])
            ow_ref[...] = jnp.where(w, pick(bw), ow_ref[...])
            oh_ref[...] = jnp.where(w, pick(bh), oh_ref[...])
            return 0

        jax.lax.fori_loop(0, _N_MAX_OBJECTS, fbody, 0)


@jax.jit
def kernel(prob, bx_dimfull, by_dimfull, bw_dimfull, bh_dimfull):
    b, n, _ = prob.shape
    n_pad = ((n + _LANE - 1) // _LANE) * _LANE

    def prep(v, fill):
        v = v[..., 0]
        return jnp.pad(v, ((0, 0), (0, n_pad - n)), constant_values=fill)

    p = prep(prob, 0.0)
    bx = prep(bx_dimfull, 0.0)
    by = prep(by_dimfull, 0.0)
    bw = prep(bw_dimfull, 1.0)
    bh = prep(bh_dimfull, 1.0)

    out = jax.ShapeDtypeStruct((b, _LANE), jnp.float32)
    ap, ax, ay, aw, ah = pl.pallas_call(
        functools.partial(_nms_body, n),
        out_shape=(out, out, out, out, out),
        scratch_shapes=[
            pltpu.VMEM((b, n_pad), jnp.float32),
            pltpu.VMEM((b, n_pad), jnp.float32),
        ],
    )(p, bx, by, bw, bh)

    k = min(_N_MAX_OBJECTS, n)
    return (ap[:, :k, None], ax[:, :k, None], ay[:, :k, None],
            aw[:, :k, None], ah[:, :k, None])
